# baseline (speedup 1.0000x reference)
import functools
import math

import jax
import jax.numpy as jnp
import numpy as np
from jax import lax
from jax.experimental import pallas as pl
from jax.experimental.pallas import tpu as pltpu

_ROTS = ((13, 15, 26, 6), (17, 29, 16, 24))
_TINY = np.float32(np.finfo(np.float32).tiny)
_INTMAX = np.int32(np.iinfo(np.int32).max)


def _threefry_bits(p):
    k0 = jnp.uint32(0)
    k1 = jnp.uint32(1)
    ks = (k0, k1, jnp.uint32(0x1BD11BDA) ^ k0 ^ k1)
    x0 = jnp.full_like(p, k0)
    x1 = p + k1
    for i in range(5):
        for r in _ROTS[i % 2]:
            x0 = x0 + x1
            x1 = (x1 << jnp.uint32(r)) | (x1 >> jnp.uint32(32 - r))
            x1 = x0 ^ x1
        x0 = x0 + ks[(i + 1) % 3]
        x1 = x1 + ks[(i + 2) % 3] + jnp.uint32(i + 1)
    return x0 ^ x1


def _gumbel(bits):
    fb = (bits >> jnp.uint32(9)) | jnp.uint32(0x3F800000)
    f = lax.bitcast_convert_type(fb, jnp.float32) - jnp.float32(1.0)
    u = jnp.maximum(f + _TINY, _TINY)
    return -jnp.log(-jnp.log(u))


def _fused_kernel(t_ref, x_ref, out_ref, cols, basep, yrun, iyrun, xrun, ixrun,
                  *, blk, ncb, vocab):
    i = pl.program_id(0)

    @pl.when(i == 0)
    def _init():
        cols[...] = lax.broadcasted_iota(jnp.int32, cols.shape, 1)
        basep[...] = (
            lax.broadcasted_iota(jnp.uint32, basep.shape, 0) * jnp.uint32(vocab)
            + lax.broadcasted_iota(jnp.uint32, basep.shape, 1))
        yrun[...] = jnp.full_like(yrun, -jnp.inf)
        iyrun[...] = jnp.zeros_like(iyrun)
        xrun[...] = jnp.full_like(xrun, -jnp.inf)
        ixrun[...] = jnp.zeros_like(ixrun)

    x = x_ref[...]
    safe_t = jnp.maximum(t_ref[...], jnp.float32(1e-6))

    def _step(masked):
        col = cols[...]
        p = basep[...] + (i * blk).astype(jnp.uint32)
        g = _gumbel(_threefry_bits(p))
        y = x / safe_t + g
        xv = x
        if masked:
            valid = col < (vocab - i * blk)
            y = jnp.where(valid, y, -jnp.inf)
            xv = jnp.where(valid, x, -jnp.inf)

        bmy = jnp.max(y, axis=1, keepdims=True)
        updy = bmy > yrun[...]

        @pl.when(jnp.any(updy))
        def _upd_y():
            biy = jnp.min(jnp.where(y == bmy, col, _INTMAX), axis=1,
                          keepdims=True) + i * blk
            iyrun[...] = jnp.where(updy, biy, iyrun[...])
            yrun[...] = jnp.where(updy, bmy, yrun[...])

        bmx = jnp.max(xv, axis=1, keepdims=True)
        updx = bmx > xrun[...]

        @pl.when(jnp.any(updx))
        def _upd_x():
            bix = jnp.min(jnp.where(xv == bmx, col, _INTMAX), axis=1,
                          keepdims=True) + i * blk
            ixrun[...] = jnp.where(updx, bix, ixrun[...])
            xrun[...] = jnp.where(updx, bmx, xrun[...])

    @pl.when(i < ncb - 1)
    def _main():
        _step(False)

    @pl.when(i == ncb - 1)
    def _last():
        _step(True)
        out_ref[...] = jnp.where(t_ref[...] <= jnp.float32(1e-6),
                                 ixrun[...], iyrun[...])


@functools.partial(jax.jit, static_argnames=("blk",))
def _sample(logits, temperatures, blk=16384):
    rows, vocab = logits.shape
    ncb = math.ceil(vocab / blk)
    t2 = temperatures.reshape(rows, 1)

    out = pl.pallas_call(
        functools.partial(_fused_kernel, blk=blk, ncb=ncb, vocab=vocab),
        grid=(ncb,),
        in_specs=[pl.BlockSpec((rows, 1), lambda i: (0, 0)),
                  pl.BlockSpec((rows, blk), lambda i: (0, i))],
        out_specs=pl.BlockSpec((rows, 1), lambda i: (0, 0)),
        out_shape=jax.ShapeDtypeStruct((rows, 1), jnp.int32),
        scratch_shapes=[pltpu.VMEM((rows, blk), jnp.int32),
                        pltpu.VMEM((rows, blk), jnp.uint32),
                        pltpu.VMEM((rows, 1), jnp.float32),
                        pltpu.VMEM((rows, 1), jnp.int32),
                        pltpu.VMEM((rows, 1), jnp.float32),
                        pltpu.VMEM((rows, 1), jnp.int32)],
    )(t2, logits)

    return out.reshape(rows)


def kernel(logits, temperatures):
    if logits.ndim == 1:
        logits = logits[None, :]
    temperatures = jnp.reshape(temperatures, (-1,))
    if temperatures.shape[0] == 1 and logits.shape[0] > 1:
        temperatures = jnp.repeat(temperatures, logits.shape[0])
    return _sample(logits, temperatures)


# speedup vs baseline: 1.0057x; 1.0057x over previous
import functools
import math

import jax
import jax.numpy as jnp
import numpy as np
from jax import lax
from jax.experimental import pallas as pl
from jax.experimental.pallas import tpu as pltpu

_ROTS = ((13, 15, 26, 6), (17, 29, 16, 24))
_TINY = np.float32(np.finfo(np.float32).tiny)
_INTMAX = np.int32(np.iinfo(np.int32).max)


def _threefry_bits(p):
    k0 = jnp.uint32(0)
    k1 = jnp.uint32(1)
    ks = (k0, k1, jnp.uint32(0x1BD11BDA) ^ k0 ^ k1)
    x0 = jnp.full_like(p, k0)
    x1 = p + k1
    for i in range(5):
        for r in _ROTS[i % 2]:
            x0 = x0 + x1
            x1 = (x1 << jnp.uint32(r)) | (x1 >> jnp.uint32(32 - r))
            x1 = x0 ^ x1
        x0 = x0 + ks[(i + 1) % 3]
        x1 = x1 + ks[(i + 2) % 3] + jnp.uint32(i + 1)
    return x0 ^ x1


def _gumbel(bits):
    fb = (bits >> jnp.uint32(9)) | jnp.uint32(0x3F800000)
    f = lax.bitcast_convert_type(fb, jnp.float32) - jnp.float32(1.0)
    u = jnp.maximum(f + _TINY, _TINY)
    return -jnp.log(-jnp.log(u))


def _fused_kernel(t_ref, x_ref, out_ref, yrun, iyrun, xrun, ixrun,
                  *, blk, ncb, vocab):
    i = pl.program_id(0)

    @pl.when(i == 0)
    def _init():
        yrun[...] = jnp.full_like(yrun, -jnp.inf)
        iyrun[...] = jnp.zeros_like(iyrun)
        xrun[...] = jnp.full_like(xrun, -jnp.inf)
        ixrun[...] = jnp.zeros_like(ixrun)

    x = x_ref[...]
    safe_t = jnp.maximum(t_ref[...], jnp.float32(1e-6))

    def _step(masked):
        col = lax.broadcasted_iota(jnp.int32, x.shape, 1) + i * blk
        p = col.astype(jnp.uint32) + (
            lax.broadcasted_iota(jnp.uint32, x.shape, 0) * jnp.uint32(vocab))
        g = _gumbel(_threefry_bits(p))
        y = x / safe_t + g
        xv = x
        if masked:
            valid = col < vocab
            y = jnp.where(valid, y, -jnp.inf)
            xv = jnp.where(valid, x, -jnp.inf)

        bmy = jnp.max(y, axis=1, keepdims=True)
        biy = jnp.min(jnp.where(y == bmy, col, _INTMAX), axis=1, keepdims=True)
        updy = bmy > yrun[...]
        iyrun[...] = jnp.where(updy, biy, iyrun[...])
        yrun[...] = jnp.where(updy, bmy, yrun[...])

        bmx = jnp.max(xv, axis=1, keepdims=True)
        bix = jnp.min(jnp.where(xv == bmx, col, _INTMAX), axis=1, keepdims=True)
        updx = bmx > xrun[...]
        ixrun[...] = jnp.where(updx, bix, ixrun[...])
        xrun[...] = jnp.where(updx, bmx, xrun[...])

    @pl.when(i < ncb - 1)
    def _main():
        _step(False)

    @pl.when(i == ncb - 1)
    def _last():
        _step(True)
        out_ref[...] = jnp.where(t_ref[...] <= jnp.float32(1e-6),
                                 ixrun[...], iyrun[...])


@functools.partial(jax.jit, static_argnames=("blk",))
def _sample(logits, temperatures, blk=16384):
    rows, vocab = logits.shape
    ncb = math.ceil(vocab / blk)
    t2 = temperatures.reshape(rows, 1)

    out = pl.pallas_call(
        functools.partial(_fused_kernel, blk=blk, ncb=ncb, vocab=vocab),
        grid=(ncb,),
        in_specs=[pl.BlockSpec((rows, 1), lambda i: (0, 0)),
                  pl.BlockSpec((rows, blk), lambda i: (0, i))],
        out_specs=pl.BlockSpec((rows, 1), lambda i: (0, 0)),
        out_shape=jax.ShapeDtypeStruct((rows, 1), jnp.int32),
        scratch_shapes=[pltpu.VMEM((rows, 1), jnp.float32),
                        pltpu.VMEM((rows, 1), jnp.int32),
                        pltpu.VMEM((rows, 1), jnp.float32),
                        pltpu.VMEM((rows, 1), jnp.int32)],
    )(t2, logits)

    return out.reshape(rows)


def kernel(logits, temperatures):
    if logits.ndim == 1:
        logits = logits[None, :]
    temperatures = jnp.reshape(temperatures, (-1,))
    if temperatures.shape[0] == 1 and logits.shape[0] > 1:
        temperatures = jnp.repeat(temperatures, logits.shape[0])
    return _sample(logits, temperatures)


# speedup vs baseline: 1.6974x; 1.6877x over previous
import functools
import math

import jax
import jax.numpy as jnp
import numpy as np
from jax import lax
from jax.experimental import pallas as pl
from jax.experimental.pallas import tpu as pltpu

_ROTS = ((13, 15, 26, 6), (17, 29, 16, 24))
_TINY = np.float32(np.finfo(np.float32).tiny)
_INTMAX = np.int32(np.iinfo(np.int32).max)


def _threefry_bits(p):
    k0 = jnp.uint32(0)
    k1 = jnp.uint32(1)
    ks = (k0, k1, jnp.uint32(0x1BD11BDA) ^ k0 ^ k1)
    x0 = jnp.full_like(p, k0)
    x1 = p + k1
    for i in range(5):
        for r in _ROTS[i % 2]:
            x0 = x0 + x1
            x1 = (x1 << jnp.uint32(r)) | (x1 >> jnp.uint32(32 - r))
            x1 = x0 ^ x1
        x0 = x0 + ks[(i + 1) % 3]
        x1 = x1 + ks[(i + 2) % 3] + jnp.uint32(i + 1)
    return x0 ^ x1


def _gumbel(bits):
    fb = (bits >> jnp.uint32(9)) | jnp.uint32(0x3F800000)
    f = lax.bitcast_convert_type(fb, jnp.float32) - jnp.float32(1.0)
    u = jnp.maximum(f + _TINY, _TINY)
    return -jnp.log(-jnp.log(u))


def _fused_kernel(t_ref, x_ref, out_ref, yrun, iyrun, xrun, ixrun,
                  *, blk, ncb, vocab):
    i = pl.program_id(0)

    @pl.when(i == 0)
    def _init():
        yrun[...] = jnp.full_like(yrun, -jnp.inf)
        iyrun[...] = jnp.zeros_like(iyrun)
        xrun[...] = jnp.full_like(xrun, -jnp.inf)
        ixrun[...] = jnp.zeros_like(ixrun)

    x = x_ref[...]
    safe_t = jnp.maximum(t_ref[...], jnp.float32(1e-6))

    col = lax.broadcasted_iota(jnp.int32, x.shape, 1) + i * blk
    p = col.astype(jnp.uint32) + (
        lax.broadcasted_iota(jnp.uint32, x.shape, 0) * jnp.uint32(vocab))
    g = _gumbel(_threefry_bits(p))
    valid = col < vocab
    y = jnp.where(valid, x / safe_t + g, -jnp.inf)
    xv = jnp.where(valid, x, -jnp.inf)

    bmy = jnp.max(y, axis=1, keepdims=True)
    biy = jnp.min(jnp.where(y == bmy, col, _INTMAX), axis=1, keepdims=True)
    updy = bmy > yrun[...]
    iyrun[...] = jnp.where(updy, biy, iyrun[...])
    yrun[...] = jnp.where(updy, bmy, yrun[...])

    bmx = jnp.max(xv, axis=1, keepdims=True)
    bix = jnp.min(jnp.where(xv == bmx, col, _INTMAX), axis=1, keepdims=True)
    updx = bmx > xrun[...]
    ixrun[...] = jnp.where(updx, bix, ixrun[...])
    xrun[...] = jnp.where(updx, bmx, xrun[...])

    @pl.when(i == ncb - 1)
    def _last():
        out_ref[...] = jnp.where(t_ref[...] <= jnp.float32(1e-6),
                                 ixrun[...], iyrun[...])


@functools.partial(jax.jit, static_argnames=("blk",))
def _sample(logits, temperatures, blk=16384):
    rows, vocab = logits.shape
    ncb = math.ceil(vocab / blk)
    t2 = temperatures.reshape(rows, 1)

    out = pl.pallas_call(
        functools.partial(_fused_kernel, blk=blk, ncb=ncb, vocab=vocab),
        grid=(ncb,),
        in_specs=[pl.BlockSpec((rows, 1), lambda i: (0, 0)),
                  pl.BlockSpec((rows, blk), lambda i: (0, i))],
        out_specs=pl.BlockSpec((rows, 1), lambda i: (0, 0)),
        out_shape=jax.ShapeDtypeStruct((rows, 1), jnp.int32),
        scratch_shapes=[pltpu.VMEM((rows, 1), jnp.float32),
                        pltpu.VMEM((rows, 1), jnp.int32),
                        pltpu.VMEM((rows, 1), jnp.float32),
                        pltpu.VMEM((rows, 1), jnp.int32)],
    )(t2, logits)

    return out.reshape(rows)


def kernel(logits, temperatures):
    if logits.ndim == 1:
        logits = logits[None, :]
    temperatures = jnp.reshape(temperatures, (-1,))
    if temperatures.shape[0] == 1 and logits.shape[0] > 1:
        temperatures = jnp.repeat(temperatures, logits.shape[0])
    return _sample(logits, temperatures)
